# baseline (speedup 1.0000x reference)
import functools

import jax
import jax.numpy as jnp
from jax import lax
from jax.experimental import pallas as pl
from jax.experimental.pallas import tpu as pltpu
from jax.experimental.pallas import tpu_sc as plsc

N = 10000
E = 320000
D = 128
NTILES = 32
KCH = 128
GRP = 8
NGRP = 10
NCH = GRP * NGRP
ETP = NCH * KCH
EP = NTILES * ETP
NROW = 10240
RPT = NROW // 16
RCH = RPT // KCH
NRK = 10112


def _edge_kernel(ia_hbm, ib_hbm, rank_hbm, w_hbm, out_hbm,
                 s_tab, rank_tab, ra_buf, rb_buf, buf0, buf1, accbuf):
  cid = lax.axis_index("c")
  sid = lax.axis_index("s")
  wid = sid * 2 + cid
  zero16 = jnp.zeros((16,), jnp.float32)

  def _zrow(r, _):
    for u in range(8):
      buf0[r, pl.ds(u * 16, 16)] = zero16
    return 0
  lax.fori_loop(0, KCH, _zrow, 0)
  r0 = sid * RPT
  for j in range(RCH):
    pltpu.sync_copy(buf0, s_tab.at[pl.ds(r0 + j * KCH, KCH)])

  pltpu.sync_copy(rank_hbm, rank_tab)

  plsc.subcore_barrier()

  def _group(g, _):
    grow = wid * NCH + g * GRP
    pltpu.sync_copy(ia_hbm.at[pl.ds(grow, GRP)], ra_buf)
    pltpu.sync_copy(ib_hbm.at[pl.ds(grow, GRP)], rb_buf)

    def _xlate(r, _):
      for u in range(KCH // 16):
        iav = ra_buf[r, pl.ds(u * 16, 16)]
        ibv = rb_buf[r, pl.ds(u * 16, 16)]
        ra_buf[r, pl.ds(u * 16, 16)] = plsc.load_gather(rank_tab, [iav])
        rb_buf[r, pl.ds(u * 16, 16)] = plsc.load_gather(rank_tab, [ibv])
      return 0
    lax.fori_loop(0, GRP, _xlate, 0)

    for j in range(GRP):
      pltpu.sync_copy(w_hbm.at[rb_buf.at[j]], buf1)
      pltpu.sync_copy(buf1, s_tab.at[ra_buf.at[j]], add=True)
    return 0
  lax.fori_loop(0, NGRP, _group, 0)

  plsc.subcore_barrier()

  acc = tuple(zero16 for _ in range(8))
  for j in range(RCH):
    pltpu.sync_copy(w_hbm.at[pl.ds(r0 + j * KCH, KCH)], buf0)
    pltpu.sync_copy(s_tab.at[pl.ds(r0 + j * KCH, KCH)], buf1)

    def _row(r, acc):
      return tuple(
          acc[u] + buf0[r, pl.ds(u * 16, 16)] * buf1[r, pl.ds(u * 16, 16)]
          for u in range(8))
    acc = lax.fori_loop(0, KCH, _row, acc)
  for u in range(8):
    accbuf[pl.ds(u * 16, 16)] = acc[u]
  pltpu.sync_copy(accbuf, out_hbm.at[wid])


@jax.jit
def _edge_stage(ia, ib, rank_ext, wpad):
  f = pl.kernel(
      _edge_kernel,
      out_type=jax.ShapeDtypeStruct((NTILES, D), jnp.float32),
      mesh=plsc.VectorSubcoreMesh(core_axis_name="c", subcore_axis_name="s"),
      compiler_params=pltpu.CompilerParams(needs_layout_passes=False),
      scratch_types=[
          pltpu.VMEM_SHARED((NROW, D), jnp.float32),
          pltpu.VMEM((NRK,), jnp.int32),
          pltpu.VMEM((GRP, KCH), jnp.int32),
          pltpu.VMEM((GRP, KCH), jnp.int32),
          pltpu.VMEM((KCH, D), jnp.float32),
          pltpu.VMEM((KCH, D), jnp.float32),
          pltpu.VMEM((D,), jnp.float32),
      ],
  )
  return f(ia, ib, rank_ext, wpad)


def kernel(edge_index, num_nodes, node_ids_weight):
  del num_nodes
  w = node_ids_weight
  cols = edge_index[1]
  deg = jnp.zeros((N,), jnp.int32).at[cols].add(1)
  pr_argsort = jnp.argsort(deg)
  rank = jnp.zeros((N,), jnp.int32).at[pr_argsort].set(
      jnp.arange(N, dtype=jnp.int32))
  a = jnp.minimum(edge_index[0], edge_index[1])
  b = jnp.maximum(edge_index[0], edge_index[1])
  key = a * N + b
  order = jnp.argsort(key)
  ks = key[order]
  first = jnp.concatenate([jnp.array([True]), ks[1:] != ks[:-1]])
  rep = jnp.zeros((E,), bool).at[order].set(first)
  ia = jnp.where(rep, a, N)

  rank_ext = jnp.concatenate(
      [rank, jnp.full((NRK - N,), N, jnp.int32)])
  wpad = jnp.concatenate(
      [w, jnp.zeros((NROW - N, D), jnp.float32)], axis=0)
  ia_p = jnp.concatenate([ia, jnp.full((EP - E,), N, jnp.int32)])
  ib_p = jnp.concatenate([b, jnp.full((EP - E,), N, jnp.int32)])

  partial = _edge_stage(ia_p.reshape(NTILES * NCH, KCH),
                        ib_p.reshape(NTILES * NCH, KCH),
                        rank_ext, wpad)
  return jnp.sum(partial, axis=0)


# speedup vs baseline: 1.4053x; 1.4053x over previous
import functools

import jax
import jax.numpy as jnp
from jax import lax
from jax.experimental import pallas as pl
from jax.experimental.pallas import tpu as pltpu
from jax.experimental.pallas import tpu_sc as plsc

N = 10000
E = 320000
D = 128
NTILES = 32
KCH = 128
GRP = 8
NGRP = 10
NCH = GRP * NGRP
ETP = NCH * KCH
EP = NTILES * ETP
EROWS = EP // KCH
NROW = 10240
RPT = NROW // 16
RCH = RPT // KCH
NRK = 10112
K1ROWS = EROWS // 16
K1GRPS = K1ROWS // GRP
K1SLOT = K1ROWS * KCH
VRANGE = 32768
TSIZE = 100_000_000


def _prep_kernel(src_hbm, dst_hbm, ia_out, ib_out, rank_out, t_hbm,
                 hist, rank_sh, totals,
                 sbuf, dbuf, kbuf, ebuf, wbuf, ones8, deg_buf, vtab,
                 rk1d, zb, tb):
  cid = lax.axis_index("c")
  sid = lax.axis_index("s")
  izero = jnp.zeros((16,), jnp.int32)
  ione = jnp.ones((16,), jnp.int32)
  iota = lax.iota(jnp.int32, 16)

  for u in range(40):
    zb[pl.ds(u * 16, 16)] = izero
  def _initrow(r, _):
    for u in range(8):
      ones8[r, pl.ds(u * 16, 16)] = ione
    return 0
  lax.fori_loop(0, GRP, _initrow, 0)
  def _zrk(i, _):
    rk1d[pl.ds(i * 16, 16)] = izero
    return 0
  lax.fori_loop(0, NROW // 16, _zrk, 0)
  def _zvt(i, _):
    vtab[pl.ds(i * 16, 16)] = izero
    return 0
  lax.fori_loop(0, VRANGE // 16, _zvt, 0)

  @pl.when(cid == 1)
  def _():
    pltpu.sync_copy(zb, hist.at[pl.ds(sid * RPT, RPT)])
    pltpu.sync_copy(zb, rank_sh.at[pl.ds(sid * RPT, RPT)])

  plsc.subcore_barrier()

  @pl.when(cid == 0)
  def _():
    def _grp(g, _):
      grow = sid * K1ROWS + g * GRP
      pltpu.sync_copy(src_hbm.at[pl.ds(grow, GRP)], sbuf)
      pltpu.sync_copy(dst_hbm.at[pl.ds(grow, GRP)], dbuf)
      def _row(r, _):
        for u in range(8):
          s = sbuf[r, pl.ds(u * 16, 16)]
          d = dbuf[r, pl.ds(u * 16, 16)]
          a = jnp.minimum(s, d)
          b = jnp.maximum(s, d)
          slot = sid * K1SLOT + (g * GRP + r) * KCH + u * 16 + iota
          realm = slot < E
          key = jnp.where(realm, a * N + b, N)
          kbuf[r, pl.ds(u * 16, 16)] = key
          ebuf[r, pl.ds(u * 16, 16)] = slot
        return 0
      lax.fori_loop(0, GRP, _row, 0)
      for j in range(GRP):
        pltpu.sync_copy(ebuf.at[j], t_hbm.at[kbuf.at[j]])
      return 0
    lax.fori_loop(0, K1GRPS, _grp, 0)

  @pl.when(cid == 1)
  def _():
    def _grp(g, _):
      grow = sid * K1ROWS + g * GRP
      pltpu.sync_copy(dst_hbm.at[pl.ds(grow, GRP)], dbuf)
      for j in range(GRP):
        pltpu.sync_copy(ones8.at[j], hist.at[dbuf.at[j]], add=True)
      return 0
    lax.fori_loop(0, K1GRPS, _grp, 0)

  plsc.subcore_barrier()

  @pl.when(cid == 0)
  def _():
    def _grp(g, _):
      grow = sid * K1ROWS + g * GRP
      pltpu.sync_copy(src_hbm.at[pl.ds(grow, GRP)], sbuf)
      pltpu.sync_copy(dst_hbm.at[pl.ds(grow, GRP)], dbuf)
      def _rowk(r, _):
        for u in range(8):
          s = sbuf[r, pl.ds(u * 16, 16)]
          d = dbuf[r, pl.ds(u * 16, 16)]
          a = jnp.minimum(s, d)
          b = jnp.maximum(s, d)
          slot = sid * K1SLOT + (g * GRP + r) * KCH + u * 16 + iota
          realm = slot < E
          key = jnp.where(realm, a * N + b, N)
          kbuf[r, pl.ds(u * 16, 16)] = key
          ebuf[r, pl.ds(u * 16, 16)] = slot
        return 0
      lax.fori_loop(0, GRP, _rowk, 0)
      for j in range(GRP):
        pltpu.sync_copy(t_hbm.at[kbuf.at[j]], wbuf.at[j])
      def _rowo(r, _):
        for u in range(8):
          s = sbuf[r, pl.ds(u * 16, 16)]
          d = dbuf[r, pl.ds(u * 16, 16)]
          a = jnp.minimum(s, d)
          b = jnp.maximum(s, d)
          slot = sid * K1SLOT + (g * GRP + r) * KCH + u * 16 + iota
          realm = slot < E
          win = wbuf[r, pl.ds(u * 16, 16)]
          rep = (win == ebuf[r, pl.ds(u * 16, 16)]) & realm
          sbuf[r, pl.ds(u * 16, 16)] = jnp.where(rep, a, N)
          dbuf[r, pl.ds(u * 16, 16)] = jnp.where(realm, b, N)
        return 0
      lax.fori_loop(0, GRP, _rowo, 0)
      pltpu.sync_copy(sbuf, ia_out.at[pl.ds(grow, GRP)])
      pltpu.sync_copy(dbuf, ib_out.at[pl.ds(grow, GRP)])
      return 0
    lax.fori_loop(0, K1GRPS, _grp, 0)

  @pl.when(cid == 1)
  def _():
    pltpu.sync_copy(hist, deg_buf)

    def _cnt(v, _):
      dv = deg_buf[pl.ds(v * 16, 16)]
      m = (dv >> 15) == sid
      loc = dv & (VRANGE - 1)
      occ, last = plsc.scan_count(dv, mask=m)
      lm = m & last
      cur = plsc.load_gather(vtab, [loc])
      plsc.store_scatter(vtab, [loc], cur + occ, mask=lm)
      return 0
    lax.fori_loop(0, N // 16, _cnt, 0)

    def _tot(i, acc):
      return acc + vtab[pl.ds(i * 16, 16)]
    accv = lax.fori_loop(0, VRANGE // 16, _tot, izero)
    total = jnp.sum(accv, axis=0)
    tb[pl.ds(0, 16)] = jnp.full((16,), total, jnp.int32)
    pltpu.sync_copy(tb.at[pl.ds(0, 8)], totals.at[pl.ds(sid * 8, 8)])

  plsc.subcore_barrier()

  @pl.when(cid == 1)
  def _():
    pltpu.sync_copy(totals, deg_buf.at[pl.ds(0, 128)])
    base = jnp.int32(0)
    for k in range(8):
      vk = deg_buf[pl.ds(k * 16, 16)]
      tv = 2 * k + (iota >= 8).astype(jnp.int32)
      sel = (tv < sid) & ((iota & 7) == 0)
      base = base + jnp.sum(jnp.where(sel, vk, 0), axis=0)

    pltpu.sync_copy(hist, deg_buf)

    def _csum(i, carry):
      x = vtab[pl.ds(i * 16, 16)]
      cs = plsc.cumsum(x)
      vtab[pl.ds(i * 16, 16)] = cs - x + carry
      return carry + jnp.sum(x, axis=0)
    lax.fori_loop(0, VRANGE // 16, _csum, base)

    def _rnk(v, _):
      dv = deg_buf[pl.ds(v * 16, 16)]
      m = (dv >> 15) == sid
      loc = dv & (VRANGE - 1)
      occ, last = plsc.scan_count(dv, mask=m)
      lm = m & last
      cur = plsc.load_gather(vtab, [loc])
      rk = jnp.where(m, cur + occ - 1, 0)
      rk1d[pl.ds(v * 16, 16)] = rk
      plsc.store_scatter(vtab, [loc], cur + occ, mask=lm)
      return 0
    lax.fori_loop(0, N // 16, _rnk, 0)

    @pl.when(sid == 0)
    def _():
      tb[pl.ds(0, 16)] = jnp.full((16,), N, jnp.int32)
      pltpu.sync_copy(tb, rank_sh.at[pl.ds(N, 16)])

    def _mrg(j, _):
      off0 = j * KCH
      for u in range(8):
        wbuf[0, pl.ds(u * 16, 16)] = rk1d[pl.ds(off0 + u * 16, 16)]
        kbuf[0, pl.ds(u * 16, 16)] = iota + (off0 + u * 16)
      pltpu.sync_copy(wbuf.at[0], rank_sh.at[kbuf.at[0]], add=True)
      return 0
    lax.fori_loop(0, NCH, _mrg, 0)

  plsc.subcore_barrier()

  @pl.when(cid == 1)
  def _():
    seg = NRK // 16
    pltpu.sync_copy(rank_sh.at[pl.ds(sid * seg, seg)],
                    deg_buf.at[pl.ds(0, seg)])
    pltpu.sync_copy(deg_buf.at[pl.ds(0, seg)],
                    rank_out.at[pl.ds(sid * seg, seg)])


def _edge_kernel(ia_hbm, ib_hbm, rank_hbm, w_hbm, out_hbm,
                 s_tab, rank_tab, ra_buf, rb_buf, buf0, buf1, accbuf):
  cid = lax.axis_index("c")
  sid = lax.axis_index("s")
  wid = sid * 2 + cid
  zero16 = jnp.zeros((16,), jnp.float32)

  def _zrow(r, _):
    for u in range(8):
      buf0[r, pl.ds(u * 16, 16)] = zero16
    return 0
  lax.fori_loop(0, KCH, _zrow, 0)
  r0 = sid * RPT
  for j in range(RCH):
    pltpu.sync_copy(buf0, s_tab.at[pl.ds(r0 + j * KCH, KCH)])

  pltpu.sync_copy(rank_hbm, rank_tab)
  rank_tab[pl.ds(N, 16)] = jnp.full((16,), N, jnp.int32)

  plsc.subcore_barrier()

  def _group(g, _):
    grow = wid * NCH + g * GRP
    pltpu.sync_copy(ia_hbm.at[pl.ds(grow, GRP)], ra_buf)
    pltpu.sync_copy(ib_hbm.at[pl.ds(grow, GRP)], rb_buf)

    def _xlate(r, _):
      for u in range(KCH // 16):
        iav = ra_buf[r, pl.ds(u * 16, 16)]
        ibv = rb_buf[r, pl.ds(u * 16, 16)]
        rav = plsc.load_gather(rank_tab, [iav])
        rbv = plsc.load_gather(rank_tab, [ibv])
        ra_buf[r, pl.ds(u * 16, 16)] = jnp.clip(rav, 0, NROW - 1)
        rb_buf[r, pl.ds(u * 16, 16)] = jnp.clip(rbv, 0, NROW - 1)
      return 0
    lax.fori_loop(0, GRP, _xlate, 0)

    for j in range(GRP):
      pltpu.sync_copy(w_hbm.at[rb_buf.at[j]], buf1)
      pltpu.sync_copy(buf1, s_tab.at[ra_buf.at[j]], add=True)
    return 0
  lax.fori_loop(0, NGRP, _group, 0)

  plsc.subcore_barrier()

  acc = tuple(zero16 for _ in range(8))
  for j in range(RCH):
    pltpu.sync_copy(w_hbm.at[pl.ds(r0 + j * KCH, KCH)], buf0)
    pltpu.sync_copy(s_tab.at[pl.ds(r0 + j * KCH, KCH)], buf1)

    def _row(r, acc):
      return tuple(
          acc[u] + buf0[r, pl.ds(u * 16, 16)] * buf1[r, pl.ds(u * 16, 16)]
          for u in range(8))
    acc = lax.fori_loop(0, KCH, _row, acc)
  for u in range(8):
    accbuf[pl.ds(u * 16, 16)] = acc[u]
  pltpu.sync_copy(accbuf, out_hbm.at[wid])


@jax.jit
def _run(src2d, dst2d, wpad):
  mesh = plsc.VectorSubcoreMesh(core_axis_name="c", subcore_axis_name="s")
  prep = pl.kernel(
      _prep_kernel,
      out_type=(
          jax.ShapeDtypeStruct((EROWS, KCH), jnp.int32),
          jax.ShapeDtypeStruct((EROWS, KCH), jnp.int32),
          jax.ShapeDtypeStruct((NRK,), jnp.int32),
          jax.ShapeDtypeStruct((TSIZE,), jnp.int32),
      ),
      mesh=mesh,
      compiler_params=pltpu.CompilerParams(needs_layout_passes=False),
      scratch_types=[
          pltpu.VMEM_SHARED((NROW,), jnp.int32),
          pltpu.VMEM_SHARED((NROW,), jnp.int32),
          pltpu.VMEM_SHARED((128,), jnp.int32),
          pltpu.VMEM((GRP, KCH), jnp.int32),
          pltpu.VMEM((GRP, KCH), jnp.int32),
          pltpu.VMEM((GRP, KCH), jnp.int32),
          pltpu.VMEM((GRP, KCH), jnp.int32),
          pltpu.VMEM((GRP, KCH), jnp.int32),
          pltpu.VMEM((GRP, KCH), jnp.int32),
          pltpu.VMEM((NROW,), jnp.int32),
          pltpu.VMEM((VRANGE,), jnp.int32),
          pltpu.VMEM((NROW,), jnp.int32),
          pltpu.VMEM((RPT,), jnp.int32),
          pltpu.VMEM((16,), jnp.int32),
      ],
  )
  ia2d, ib2d, rank_ext, _ = prep(src2d, dst2d)

  edge = pl.kernel(
      _edge_kernel,
      out_type=jax.ShapeDtypeStruct((NTILES, D), jnp.float32),
      mesh=mesh,
      compiler_params=pltpu.CompilerParams(needs_layout_passes=False),
      scratch_types=[
          pltpu.VMEM_SHARED((NROW, D), jnp.float32),
          pltpu.VMEM((NRK,), jnp.int32),
          pltpu.VMEM((GRP, KCH), jnp.int32),
          pltpu.VMEM((GRP, KCH), jnp.int32),
          pltpu.VMEM((KCH, D), jnp.float32),
          pltpu.VMEM((KCH, D), jnp.float32),
          pltpu.VMEM((D,), jnp.float32),
      ],
  )
  partial = edge(ia2d, ib2d, rank_ext, wpad)
  return jnp.sum(partial, axis=0)


def kernel(edge_index, num_nodes, node_ids_weight):
  del num_nodes
  pad = jnp.full((EP - E,), N, jnp.int32)
  src2d = jnp.concatenate([edge_index[0], pad]).reshape(EROWS, KCH)
  dst2d = jnp.concatenate([edge_index[1], pad]).reshape(EROWS, KCH)
  wpad = jnp.concatenate(
      [node_ids_weight, jnp.zeros((NROW - N, D), jnp.float32)], axis=0)
  return _run(src2d, dst2d, wpad)


# speedup vs baseline: 1.4626x; 1.0408x over previous
import functools

import jax
import jax.numpy as jnp
from jax import lax
from jax.experimental import pallas as pl
from jax.experimental.pallas import tpu as pltpu
from jax.experimental.pallas import tpu_sc as plsc

N = 10000
E = 320000
D = 128
NTILES = 32
KCH = 128
GRP = 8
NGRP = 10
NCH = GRP * NGRP
ETP = NCH * KCH
EP = NTILES * ETP
EROWS = EP // KCH
NROW = 10240
RPT = NROW // 16
RCH = RPT // KCH
NRK = 10112
K1ROWS = EROWS // 16
K1GRPS = K1ROWS // GRP
K1SLOT = K1ROWS * KCH
VRANGE = 32768
TSIZE = 100_000_000


def _prep_kernel(src_hbm, dst_hbm, ia_out, ib_out, rank_out, t_hbm,
                 hist, rank_sh, totals,
                 sbuf, dbuf, kbuf, ebuf, wbuf, ones8, deg_buf, vtab,
                 rk1d, zb, tb, dsem):
  cid = lax.axis_index("c")
  sid = lax.axis_index("s")
  izero = jnp.zeros((16,), jnp.int32)
  ione = jnp.ones((16,), jnp.int32)
  iota = lax.iota(jnp.int32, 16)

  for u in range(40):
    zb[pl.ds(u * 16, 16)] = izero
  def _initrow(r, _):
    for u in range(8):
      ones8[r, pl.ds(u * 16, 16)] = ione
    return 0
  lax.fori_loop(0, GRP, _initrow, 0)
  def _zrk(i, _):
    rk1d[pl.ds(i * 16, 16)] = izero
    return 0
  lax.fori_loop(0, NROW // 16, _zrk, 0)
  def _zvt(i, _):
    vtab[pl.ds(i * 16, 16)] = izero
    return 0
  lax.fori_loop(0, VRANGE // 16, _zvt, 0)

  @pl.when(cid == 1)
  def _():
    pltpu.sync_copy(zb, hist.at[pl.ds(sid * RPT, RPT)])
    pltpu.sync_copy(zb, rank_sh.at[pl.ds(sid * RPT, RPT)])

  plsc.subcore_barrier()

  @pl.when(cid == 0)
  def _():
    def _grp(g, _):
      grow = sid * K1ROWS + g * GRP
      pltpu.sync_copy(src_hbm.at[pl.ds(grow, GRP)], sbuf)
      pltpu.sync_copy(dst_hbm.at[pl.ds(grow, GRP)], dbuf)
      def _row(r, _):
        for u in range(8):
          s = sbuf[r, pl.ds(u * 16, 16)]
          d = dbuf[r, pl.ds(u * 16, 16)]
          a = jnp.minimum(s, d)
          b = jnp.maximum(s, d)
          slot = sid * K1SLOT + (g * GRP + r) * KCH + u * 16 + iota
          realm = slot < E
          key = jnp.where(realm, a * N + b, N)
          kbuf[r, pl.ds(u * 16, 16)] = key
          ebuf[r, pl.ds(u * 16, 16)] = slot
        return 0
      lax.fori_loop(0, GRP, _row, 0)
      cps = [pltpu.async_copy(ebuf.at[j], t_hbm.at[kbuf.at[j]], dsem)
             for j in range(GRP)]
      for cp in cps:
        cp.wait()
      return 0
    lax.fori_loop(0, K1GRPS, _grp, 0)

  @pl.when(cid == 1)
  def _():
    def _grp(g, _):
      grow = sid * K1ROWS + g * GRP
      pltpu.sync_copy(dst_hbm.at[pl.ds(grow, GRP)], dbuf)
      cps = [pltpu.async_copy(ones8.at[j], hist.at[dbuf.at[j]], dsem,
                              add=True)
             for j in range(GRP)]
      for cp in cps:
        cp.wait()
      return 0
    lax.fori_loop(0, K1GRPS, _grp, 0)

  plsc.subcore_barrier()

  @pl.when(cid == 0)
  def _():
    def _grp(g, _):
      grow = sid * K1ROWS + g * GRP
      pltpu.sync_copy(src_hbm.at[pl.ds(grow, GRP)], sbuf)
      pltpu.sync_copy(dst_hbm.at[pl.ds(grow, GRP)], dbuf)
      def _rowk(r, _):
        for u in range(8):
          s = sbuf[r, pl.ds(u * 16, 16)]
          d = dbuf[r, pl.ds(u * 16, 16)]
          a = jnp.minimum(s, d)
          b = jnp.maximum(s, d)
          slot = sid * K1SLOT + (g * GRP + r) * KCH + u * 16 + iota
          realm = slot < E
          key = jnp.where(realm, a * N + b, N)
          kbuf[r, pl.ds(u * 16, 16)] = key
          ebuf[r, pl.ds(u * 16, 16)] = slot
        return 0
      lax.fori_loop(0, GRP, _rowk, 0)
      cps = [pltpu.async_copy(t_hbm.at[kbuf.at[j]], wbuf.at[j], dsem)
             for j in range(GRP)]
      for cp in cps:
        cp.wait()
      def _rowo(r, _):
        for u in range(8):
          s = sbuf[r, pl.ds(u * 16, 16)]
          d = dbuf[r, pl.ds(u * 16, 16)]
          a = jnp.minimum(s, d)
          b = jnp.maximum(s, d)
          slot = sid * K1SLOT + (g * GRP + r) * KCH + u * 16 + iota
          realm = slot < E
          win = wbuf[r, pl.ds(u * 16, 16)]
          rep = (win == ebuf[r, pl.ds(u * 16, 16)]) & realm
          sbuf[r, pl.ds(u * 16, 16)] = jnp.where(rep, a, N)
          dbuf[r, pl.ds(u * 16, 16)] = jnp.where(realm, b, N)
        return 0
      lax.fori_loop(0, GRP, _rowo, 0)
      pltpu.sync_copy(sbuf, ia_out.at[pl.ds(grow, GRP)])
      pltpu.sync_copy(dbuf, ib_out.at[pl.ds(grow, GRP)])
      return 0
    lax.fori_loop(0, K1GRPS, _grp, 0)

  @pl.when(cid == 1)
  def _():
    pltpu.sync_copy(hist, deg_buf)

    def _cnt(v, _):
      dv = deg_buf[pl.ds(v * 16, 16)]
      m = (dv >> 15) == sid
      loc = dv & (VRANGE - 1)
      occ, last = plsc.scan_count(dv, mask=m)
      lm = m & last
      cur = plsc.load_gather(vtab, [loc])
      plsc.store_scatter(vtab, [loc], cur + occ, mask=lm)
      return 0
    lax.fori_loop(0, N // 16, _cnt, 0)

    def _tot(i, acc):
      return acc + vtab[pl.ds(i * 16, 16)]
    accv = lax.fori_loop(0, VRANGE // 16, _tot, izero)
    total = jnp.sum(accv, axis=0)
    tb[pl.ds(0, 16)] = jnp.full((16,), total, jnp.int32)
    pltpu.sync_copy(tb.at[pl.ds(0, 8)], totals.at[pl.ds(sid * 8, 8)])

  plsc.subcore_barrier()

  @pl.when(cid == 1)
  def _():
    pltpu.sync_copy(totals, deg_buf.at[pl.ds(0, 128)])
    base = jnp.int32(0)
    for k in range(8):
      vk = deg_buf[pl.ds(k * 16, 16)]
      tv = 2 * k + (iota >= 8).astype(jnp.int32)
      sel = (tv < sid) & ((iota & 7) == 0)
      base = base + jnp.sum(jnp.where(sel, vk, 0), axis=0)

    pltpu.sync_copy(hist, deg_buf)

    def _csum(i, carry):
      x = vtab[pl.ds(i * 16, 16)]
      cs = plsc.cumsum(x)
      vtab[pl.ds(i * 16, 16)] = cs - x + carry
      return carry + jnp.sum(x, axis=0)
    lax.fori_loop(0, VRANGE // 16, _csum, base)

    def _rnk(v, _):
      dv = deg_buf[pl.ds(v * 16, 16)]
      m = (dv >> 15) == sid
      loc = dv & (VRANGE - 1)
      occ, last = plsc.scan_count(dv, mask=m)
      lm = m & last
      cur = plsc.load_gather(vtab, [loc])
      rk = jnp.where(m, cur + occ - 1, 0)
      rk1d[pl.ds(v * 16, 16)] = rk
      plsc.store_scatter(vtab, [loc], cur + occ, mask=lm)
      return 0
    lax.fori_loop(0, N // 16, _rnk, 0)

    @pl.when(sid == 0)
    def _():
      tb[pl.ds(0, 16)] = jnp.full((16,), N, jnp.int32)
      pltpu.sync_copy(tb, rank_sh.at[pl.ds(N, 16)])

    def _mgrp(g, _):
      for r2 in range(GRP):
        off0 = (g * GRP + r2) * KCH
        for u in range(8):
          wbuf[r2, pl.ds(u * 16, 16)] = rk1d[pl.ds(off0 + u * 16, 16)]
          kbuf[r2, pl.ds(u * 16, 16)] = iota + (off0 + u * 16)
      cps = [pltpu.async_copy(wbuf.at[r2], rank_sh.at[kbuf.at[r2]], dsem,
                              add=True)
             for r2 in range(GRP)]
      for cp in cps:
        cp.wait()
      return 0
    lax.fori_loop(0, NGRP, _mgrp, 0)

  plsc.subcore_barrier()

  @pl.when(cid == 1)
  def _():
    seg = NRK // 16
    pltpu.sync_copy(rank_sh.at[pl.ds(sid * seg, seg)],
                    deg_buf.at[pl.ds(0, seg)])
    pltpu.sync_copy(deg_buf.at[pl.ds(0, seg)],
                    rank_out.at[pl.ds(sid * seg, seg)])


def _edge_kernel(ia_hbm, ib_hbm, rank_hbm, w_hbm, out_hbm,
                 s_tab, rank_tab, ra_buf, rb_buf, buf0, buf1, accbuf,
                 gsem):
  cid = lax.axis_index("c")
  sid = lax.axis_index("s")
  wid = sid * 2 + cid
  zero16 = jnp.zeros((16,), jnp.float32)

  def _zrow(r, _):
    for u in range(8):
      buf0[r, pl.ds(u * 16, 16)] = zero16
    return 0
  lax.fori_loop(0, KCH, _zrow, 0)
  r0 = sid * RPT
  for j in range(RCH):
    pltpu.sync_copy(buf0, s_tab.at[pl.ds(r0 + j * KCH, KCH)])

  pltpu.sync_copy(rank_hbm, rank_tab)
  rank_tab[pl.ds(N, 16)] = jnp.full((16,), N, jnp.int32)

  plsc.subcore_barrier()

  def _group(g, _):
    grow = wid * NCH + g * GRP
    pltpu.sync_copy(ia_hbm.at[pl.ds(grow, GRP)], ra_buf)
    pltpu.sync_copy(ib_hbm.at[pl.ds(grow, GRP)], rb_buf)

    def _xlate(r, _):
      for u in range(KCH // 16):
        iav = ra_buf[r, pl.ds(u * 16, 16)]
        ibv = rb_buf[r, pl.ds(u * 16, 16)]
        rav = plsc.load_gather(rank_tab, [iav])
        rbv = plsc.load_gather(rank_tab, [ibv])
        ra_buf[r, pl.ds(u * 16, 16)] = jnp.clip(rav, 0, NROW - 1)
        rb_buf[r, pl.ds(u * 16, 16)] = jnp.clip(rbv, 0, NROW - 1)
      return 0
    lax.fori_loop(0, GRP, _xlate, 0)

    bufs = (buf0, buf1)
    cp = pltpu.async_copy(w_hbm.at[rb_buf.at[0]], bufs[0], gsem)
    for j in range(GRP):
      nxt = None
      if j + 1 < GRP:
        nxt = pltpu.async_copy(w_hbm.at[rb_buf.at[j + 1]], bufs[(j + 1) % 2],
                               gsem)
      cp.wait()
      pltpu.sync_copy(bufs[j % 2], s_tab.at[ra_buf.at[j]], add=True)
      cp = nxt
    return 0
  lax.fori_loop(0, NGRP, _group, 0)

  plsc.subcore_barrier()

  acc = tuple(zero16 for _ in range(8))
  for j in range(RCH):
    pltpu.sync_copy(w_hbm.at[pl.ds(r0 + j * KCH, KCH)], buf0)
    pltpu.sync_copy(s_tab.at[pl.ds(r0 + j * KCH, KCH)], buf1)

    def _row(r, acc):
      return tuple(
          acc[u] + buf0[r, pl.ds(u * 16, 16)] * buf1[r, pl.ds(u * 16, 16)]
          for u in range(8))
    acc = lax.fori_loop(0, KCH, _row, acc)
  for u in range(8):
    accbuf[pl.ds(u * 16, 16)] = acc[u]
  pltpu.sync_copy(accbuf, out_hbm.at[wid])


@jax.jit
def _run(src2d, dst2d, wpad):
  mesh = plsc.VectorSubcoreMesh(core_axis_name="c", subcore_axis_name="s")
  prep = pl.kernel(
      _prep_kernel,
      out_type=(
          jax.ShapeDtypeStruct((EROWS, KCH), jnp.int32),
          jax.ShapeDtypeStruct((EROWS, KCH), jnp.int32),
          jax.ShapeDtypeStruct((NRK,), jnp.int32),
          jax.ShapeDtypeStruct((TSIZE,), jnp.int32),
      ),
      mesh=mesh,
      compiler_params=pltpu.CompilerParams(needs_layout_passes=False),
      scratch_types=[
          pltpu.VMEM_SHARED((NROW,), jnp.int32),
          pltpu.VMEM_SHARED((NROW,), jnp.int32),
          pltpu.VMEM_SHARED((128,), jnp.int32),
          pltpu.VMEM((GRP, KCH), jnp.int32),
          pltpu.VMEM((GRP, KCH), jnp.int32),
          pltpu.VMEM((GRP, KCH), jnp.int32),
          pltpu.VMEM((GRP, KCH), jnp.int32),
          pltpu.VMEM((GRP, KCH), jnp.int32),
          pltpu.VMEM((GRP, KCH), jnp.int32),
          pltpu.VMEM((NROW,), jnp.int32),
          pltpu.VMEM((VRANGE,), jnp.int32),
          pltpu.VMEM((NROW,), jnp.int32),
          pltpu.VMEM((RPT,), jnp.int32),
          pltpu.VMEM((16,), jnp.int32),
          pltpu.SemaphoreType.DMA,
      ],
  )
  ia2d, ib2d, rank_ext, _ = prep(src2d, dst2d)

  edge = pl.kernel(
      _edge_kernel,
      out_type=jax.ShapeDtypeStruct((NTILES, D), jnp.float32),
      mesh=mesh,
      compiler_params=pltpu.CompilerParams(needs_layout_passes=False),
      scratch_types=[
          pltpu.VMEM_SHARED((NROW, D), jnp.float32),
          pltpu.VMEM((NRK,), jnp.int32),
          pltpu.VMEM((GRP, KCH), jnp.int32),
          pltpu.VMEM((GRP, KCH), jnp.int32),
          pltpu.VMEM((KCH, D), jnp.float32),
          pltpu.VMEM((KCH, D), jnp.float32),
          pltpu.VMEM((D,), jnp.float32),
          pltpu.SemaphoreType.DMA,
      ],
  )
  partial = edge(ia2d, ib2d, rank_ext, wpad)
  return jnp.sum(partial, axis=0)


def kernel(edge_index, num_nodes, node_ids_weight):
  del num_nodes
  pad = jnp.full((EP - E,), N, jnp.int32)
  src2d = jnp.concatenate([edge_index[0], pad]).reshape(EROWS, KCH)
  dst2d = jnp.concatenate([edge_index[1], pad]).reshape(EROWS, KCH)
  wpad = jnp.concatenate(
      [node_ids_weight, jnp.zeros((NROW - N, D), jnp.float32)], axis=0)
  return _run(src2d, dst2d, wpad)


# speedup vs baseline: 1.4661x; 1.0024x over previous
import functools

import jax
import jax.numpy as jnp
from jax import lax
from jax.experimental import pallas as pl
from jax.experimental.pallas import tpu as pltpu
from jax.experimental.pallas import tpu_sc as plsc

N = 10000
E = 320000
D = 128
NTILES = 32
KCH = 128
GRP = 8
NGRP = 10
NCH = GRP * NGRP
ETP = NCH * KCH
EP = NTILES * ETP
EROWS = EP // KCH
NROW = 10240
RPT = NROW // 16
RCH = RPT // KCH
NRK = 10112
K1ROWS = EROWS // 16
K1G = 16
K1GRPS = K1ROWS // K1G
K1SLOT = K1ROWS * KCH
VRANGE = 32768
TSIZE = 100_000_000


def _prep_kernel(src_hbm, dst_hbm, ia_out, ib_out, rank_out, t_hbm,
                 hist, rank_sh, totals,
                 sbuf, dbuf, kbuf, ebuf, wbuf, ones8, deg_buf, vtab,
                 rk1d, zb, tb, dsem):
  cid = lax.axis_index("c")
  sid = lax.axis_index("s")
  izero = jnp.zeros((16,), jnp.int32)
  ione = jnp.ones((16,), jnp.int32)
  iota = lax.iota(jnp.int32, 16)

  for u in range(40):
    zb[pl.ds(u * 16, 16)] = izero
  def _initrow(r, _):
    for u in range(8):
      ones8[r, pl.ds(u * 16, 16)] = ione
    return 0
  lax.fori_loop(0, K1G, _initrow, 0)
  def _zrk(i, _):
    rk1d[pl.ds(i * 16, 16)] = izero
    return 0
  lax.fori_loop(0, NROW // 16, _zrk, 0)
  def _zvt(i, _):
    vtab[pl.ds(i * 16, 16)] = izero
    return 0
  lax.fori_loop(0, VRANGE // 16, _zvt, 0)

  @pl.when(cid == 1)
  def _():
    pltpu.sync_copy(zb, hist.at[pl.ds(sid * RPT, RPT)])
    pltpu.sync_copy(zb, rank_sh.at[pl.ds(sid * RPT, RPT)])

  plsc.subcore_barrier()

  @pl.when(cid == 0)
  def _():
    def _grp(g, _):
      grow = sid * K1ROWS + g * K1G
      pltpu.sync_copy(src_hbm.at[pl.ds(grow, K1G)], sbuf)
      pltpu.sync_copy(dst_hbm.at[pl.ds(grow, K1G)], dbuf)
      def _row(r, _):
        for u in range(8):
          s = sbuf[r, pl.ds(u * 16, 16)]
          d = dbuf[r, pl.ds(u * 16, 16)]
          a = jnp.minimum(s, d)
          b = jnp.maximum(s, d)
          slot = sid * K1SLOT + (g * K1G + r) * KCH + u * 16 + iota
          realm = slot < E
          key = jnp.where(realm, a * N + b, N)
          kbuf[r, pl.ds(u * 16, 16)] = key
          ebuf[r, pl.ds(u * 16, 16)] = slot
        return 0
      lax.fori_loop(0, K1G, _row, 0)
      cps = [pltpu.async_copy(ebuf.at[j], t_hbm.at[kbuf.at[j]], dsem)
             for j in range(K1G)]
      for cp in cps:
        cp.wait()
      return 0
    lax.fori_loop(0, K1GRPS, _grp, 0)

  @pl.when(cid == 1)
  def _():
    def _grp(g, _):
      grow = sid * K1ROWS + g * K1G
      pltpu.sync_copy(dst_hbm.at[pl.ds(grow, K1G)], dbuf)
      cps = [pltpu.async_copy(ones8.at[j], hist.at[dbuf.at[j]], dsem,
                              add=True)
             for j in range(K1G)]
      for cp in cps:
        cp.wait()
      return 0
    lax.fori_loop(0, K1GRPS, _grp, 0)

  plsc.subcore_barrier()

  @pl.when(cid == 0)
  def _():
    def _grp(g, _):
      grow = sid * K1ROWS + g * K1G
      pltpu.sync_copy(src_hbm.at[pl.ds(grow, K1G)], sbuf)
      pltpu.sync_copy(dst_hbm.at[pl.ds(grow, K1G)], dbuf)
      def _rowk(r, _):
        for u in range(8):
          s = sbuf[r, pl.ds(u * 16, 16)]
          d = dbuf[r, pl.ds(u * 16, 16)]
          a = jnp.minimum(s, d)
          b = jnp.maximum(s, d)
          slot = sid * K1SLOT + (g * K1G + r) * KCH + u * 16 + iota
          realm = slot < E
          key = jnp.where(realm, a * N + b, N)
          kbuf[r, pl.ds(u * 16, 16)] = key
          ebuf[r, pl.ds(u * 16, 16)] = slot
        return 0
      lax.fori_loop(0, K1G, _rowk, 0)
      cps = [pltpu.async_copy(t_hbm.at[kbuf.at[j]], wbuf.at[j], dsem)
             for j in range(K1G)]
      for cp in cps:
        cp.wait()
      def _rowo(r, _):
        for u in range(8):
          s = sbuf[r, pl.ds(u * 16, 16)]
          d = dbuf[r, pl.ds(u * 16, 16)]
          a = jnp.minimum(s, d)
          b = jnp.maximum(s, d)
          slot = sid * K1SLOT + (g * K1G + r) * KCH + u * 16 + iota
          realm = slot < E
          win = wbuf[r, pl.ds(u * 16, 16)]
          rep = (win == ebuf[r, pl.ds(u * 16, 16)]) & realm
          sbuf[r, pl.ds(u * 16, 16)] = jnp.where(rep, a, N)
          dbuf[r, pl.ds(u * 16, 16)] = jnp.where(realm, b, N)
        return 0
      lax.fori_loop(0, K1G, _rowo, 0)
      pltpu.sync_copy(sbuf, ia_out.at[pl.ds(grow, K1G)])
      pltpu.sync_copy(dbuf, ib_out.at[pl.ds(grow, K1G)])
      return 0
    lax.fori_loop(0, K1GRPS, _grp, 0)

  @pl.when(cid == 1)
  def _():
    pltpu.sync_copy(hist, deg_buf)

    def _cnt(v, _):
      dv = deg_buf[pl.ds(v * 16, 16)]
      m = (dv >> 15) == sid
      loc = dv & (VRANGE - 1)
      occ, last = plsc.scan_count(dv, mask=m)
      lm = m & last
      cur = plsc.load_gather(vtab, [loc])
      plsc.store_scatter(vtab, [loc], cur + occ, mask=lm)
      return 0
    lax.fori_loop(0, N // 16, _cnt, 0)

    def _tot(i, acc):
      return acc + vtab[pl.ds(i * 16, 16)]
    accv = lax.fori_loop(0, VRANGE // 16, _tot, izero)
    total = jnp.sum(accv, axis=0)
    tb[pl.ds(0, 16)] = jnp.full((16,), total, jnp.int32)
    pltpu.sync_copy(tb.at[pl.ds(0, 8)], totals.at[pl.ds(sid * 8, 8)])

  plsc.subcore_barrier()

  @pl.when(cid == 1)
  def _():
    pltpu.sync_copy(totals, deg_buf.at[pl.ds(0, 128)])
    base = jnp.int32(0)
    for k in range(8):
      vk = deg_buf[pl.ds(k * 16, 16)]
      tv = 2 * k + (iota >= 8).astype(jnp.int32)
      sel = (tv < sid) & ((iota & 7) == 0)
      base = base + jnp.sum(jnp.where(sel, vk, 0), axis=0)

    pltpu.sync_copy(hist, deg_buf)

    def _csum(i, carry):
      x = vtab[pl.ds(i * 16, 16)]
      cs = plsc.cumsum(x)
      vtab[pl.ds(i * 16, 16)] = cs - x + carry
      return carry + jnp.sum(x, axis=0)
    lax.fori_loop(0, VRANGE // 16, _csum, base)

    def _rnk(v, _):
      dv = deg_buf[pl.ds(v * 16, 16)]
      m = (dv >> 15) == sid
      loc = dv & (VRANGE - 1)
      occ, last = plsc.scan_count(dv, mask=m)
      lm = m & last
      cur = plsc.load_gather(vtab, [loc])
      rk = jnp.where(m, cur + occ - 1, 0)
      rk1d[pl.ds(v * 16, 16)] = rk
      plsc.store_scatter(vtab, [loc], cur + occ, mask=lm)
      return 0
    lax.fori_loop(0, N // 16, _rnk, 0)

    @pl.when(sid == 0)
    def _():
      tb[pl.ds(0, 16)] = jnp.full((16,), N, jnp.int32)
      pltpu.sync_copy(tb, rank_sh.at[pl.ds(N, 16)])

    def _mgrp(g, _):
      for r2 in range(K1G):
        off0 = (g * K1G + r2) * KCH
        for u in range(8):
          wbuf[r2, pl.ds(u * 16, 16)] = rk1d[pl.ds(off0 + u * 16, 16)]
          kbuf[r2, pl.ds(u * 16, 16)] = iota + (off0 + u * 16)
      cps = [pltpu.async_copy(wbuf.at[r2], rank_sh.at[kbuf.at[r2]], dsem,
                              add=True)
             for r2 in range(K1G)]
      for cp in cps:
        cp.wait()
      return 0
    lax.fori_loop(0, NCH // K1G, _mgrp, 0)

  plsc.subcore_barrier()

  @pl.when(cid == 1)
  def _():
    seg = NRK // 16
    pltpu.sync_copy(rank_sh.at[pl.ds(sid * seg, seg)],
                    deg_buf.at[pl.ds(0, seg)])
    pltpu.sync_copy(deg_buf.at[pl.ds(0, seg)],
                    rank_out.at[pl.ds(sid * seg, seg)])


def _edge_kernel(ia_hbm, ib_hbm, rank_hbm, w_hbm, out_hbm,
                 s_tab, rank_tab, ra_buf, rb_buf, buf0, buf1, accbuf,
                 gsem):
  cid = lax.axis_index("c")
  sid = lax.axis_index("s")
  wid = sid * 2 + cid
  zero16 = jnp.zeros((16,), jnp.float32)

  def _zrow(r, _):
    for u in range(8):
      buf0[r, pl.ds(u * 16, 16)] = zero16
    return 0
  lax.fori_loop(0, KCH, _zrow, 0)
  r0 = sid * RPT
  for j in range(RCH):
    pltpu.sync_copy(buf0, s_tab.at[pl.ds(r0 + j * KCH, KCH)])

  pltpu.sync_copy(rank_hbm, rank_tab)
  rank_tab[pl.ds(N, 16)] = jnp.full((16,), N, jnp.int32)

  plsc.subcore_barrier()

  def _group(g, _):
    grow = wid * NCH + g * GRP
    pltpu.sync_copy(ia_hbm.at[pl.ds(grow, GRP)], ra_buf)
    pltpu.sync_copy(ib_hbm.at[pl.ds(grow, GRP)], rb_buf)

    def _xlate(r, _):
      for u in range(KCH // 16):
        iav = ra_buf[r, pl.ds(u * 16, 16)]
        ibv = rb_buf[r, pl.ds(u * 16, 16)]
        rav = plsc.load_gather(rank_tab, [iav])
        rbv = plsc.load_gather(rank_tab, [ibv])
        ra_buf[r, pl.ds(u * 16, 16)] = jnp.clip(rav, 0, NROW - 1)
        rb_buf[r, pl.ds(u * 16, 16)] = jnp.clip(rbv, 0, NROW - 1)
      return 0
    lax.fori_loop(0, GRP, _xlate, 0)

    bufs = (buf0, buf1)
    cp = pltpu.async_copy(w_hbm.at[rb_buf.at[0]], bufs[0], gsem)
    for j in range(GRP):
      nxt = None
      if j + 1 < GRP:
        nxt = pltpu.async_copy(w_hbm.at[rb_buf.at[j + 1]], bufs[(j + 1) % 2],
                               gsem)
      cp.wait()
      pltpu.sync_copy(bufs[j % 2], s_tab.at[ra_buf.at[j]], add=True)
      cp = nxt
    return 0
  lax.fori_loop(0, NGRP, _group, 0)

  plsc.subcore_barrier()

  acc = tuple(zero16 for _ in range(8))
  for j in range(RCH):
    pltpu.sync_copy(w_hbm.at[pl.ds(r0 + j * KCH, KCH)], buf0)
    pltpu.sync_copy(s_tab.at[pl.ds(r0 + j * KCH, KCH)], buf1)

    def _row(r, acc):
      return tuple(
          acc[u] + buf0[r, pl.ds(u * 16, 16)] * buf1[r, pl.ds(u * 16, 16)]
          for u in range(8))
    acc = lax.fori_loop(0, KCH, _row, acc)
  for u in range(8):
    accbuf[pl.ds(u * 16, 16)] = acc[u]
  pltpu.sync_copy(accbuf, out_hbm.at[wid])


@jax.jit
def _run(src2d, dst2d, wpad):
  mesh = plsc.VectorSubcoreMesh(core_axis_name="c", subcore_axis_name="s")
  prep = pl.kernel(
      _prep_kernel,
      out_type=(
          jax.ShapeDtypeStruct((EROWS, KCH), jnp.int32),
          jax.ShapeDtypeStruct((EROWS, KCH), jnp.int32),
          jax.ShapeDtypeStruct((NRK,), jnp.int32),
          jax.ShapeDtypeStruct((TSIZE,), jnp.int32),
      ),
      mesh=mesh,
      compiler_params=pltpu.CompilerParams(needs_layout_passes=False),
      scratch_types=[
          pltpu.VMEM_SHARED((NROW,), jnp.int32),
          pltpu.VMEM_SHARED((NROW,), jnp.int32),
          pltpu.VMEM_SHARED((128,), jnp.int32),
          pltpu.VMEM((K1G, KCH), jnp.int32),
          pltpu.VMEM((K1G, KCH), jnp.int32),
          pltpu.VMEM((K1G, KCH), jnp.int32),
          pltpu.VMEM((K1G, KCH), jnp.int32),
          pltpu.VMEM((K1G, KCH), jnp.int32),
          pltpu.VMEM((K1G, KCH), jnp.int32),
          pltpu.VMEM((NROW,), jnp.int32),
          pltpu.VMEM((VRANGE,), jnp.int32),
          pltpu.VMEM((NROW,), jnp.int32),
          pltpu.VMEM((RPT,), jnp.int32),
          pltpu.VMEM((16,), jnp.int32),
          pltpu.SemaphoreType.DMA,
      ],
  )
  ia2d, ib2d, rank_ext, _ = prep(src2d, dst2d)

  edge = pl.kernel(
      _edge_kernel,
      out_type=jax.ShapeDtypeStruct((NTILES, D), jnp.float32),
      mesh=mesh,
      compiler_params=pltpu.CompilerParams(needs_layout_passes=False),
      scratch_types=[
          pltpu.VMEM_SHARED((NROW, D), jnp.float32),
          pltpu.VMEM((NRK,), jnp.int32),
          pltpu.VMEM((GRP, KCH), jnp.int32),
          pltpu.VMEM((GRP, KCH), jnp.int32),
          pltpu.VMEM((KCH, D), jnp.float32),
          pltpu.VMEM((KCH, D), jnp.float32),
          pltpu.VMEM((D,), jnp.float32),
          pltpu.SemaphoreType.DMA,
      ],
  )
  partial = edge(ia2d, ib2d, rank_ext, wpad)
  return jnp.sum(partial, axis=0)


def kernel(edge_index, num_nodes, node_ids_weight):
  del num_nodes
  pad = jnp.full((EP - E,), N, jnp.int32)
  src2d = jnp.concatenate([edge_index[0], pad]).reshape(EROWS, KCH)
  dst2d = jnp.concatenate([edge_index[1], pad]).reshape(EROWS, KCH)
  wpad = jnp.concatenate(
      [node_ids_weight, jnp.zeros((NROW - N, D), jnp.float32)], axis=0)
  return _run(src2d, dst2d, wpad)
